# baseline (speedup 1.0000x reference)
import functools

import numpy as np
import jax
import jax.numpy as jnp
from jax import lax
from jax.experimental import pallas as pl
from jax.experimental.pallas import tpu as pltpu
from jax.experimental.pallas import tpu_sc as plsc

_N_LEVELS = 16
_LOG2_T = 19
_T = 1 << _LOG2_T
_TMASK = _T - 1
_BASE_RES = 16
_FINE_RES = 2048
_N = 262144
_SCALE = float(np.exp(np.log(_FINE_RES / _BASE_RES) / (_N_LEVELS - 1)))
_RES = [int(np.floor(_BASE_RES * _SCALE ** l)) for l in range(_N_LEVELS)]
_P2 = 2654435761
_P3 = 805459861

_NC, _NS, _LANES = 2, 16, 16
_NW = _NC * _NS
_C = 256
_G = _C // _LANES
_PW = _N // _NW
_CHUNKS = _PW // _C

_F = 2 * _N_LEVELS + 4
_M = _N_LEVELS * _T * 2
_PT = _M // _NW
_RB = 8192


def _repack_body(tin_hbm, tout_hbm, bins, bouts, sin, sout):
    wid = lax.axis_index("s") * _NC + lax.axis_index("c")
    base = wid * _PT
    iota8 = lax.iota(jnp.int32, _LANES) * 8
    n_it = _PT // _RB

    def interleave(d, bout):
        def tp(k, c2):
            for q in range(8):
                f0 = d[pl.ds(k * 256 + q * 16, 16)]
                f1 = d[pl.ds(k * 256 + 128 + q * 16, 16)]
                plsc.store_scatter(bout, [(k * 128 + q * 16) * 8 + iota8], f0)
                plsc.store_scatter(bout, [(k * 128 + q * 16) * 8 + 1 + iota8], f1)
            return c2

        lax.fori_loop(0, _RB // 256, tp, 0)

    ins = [None, None]
    outs = [None, None]
    ins[0] = pltpu.async_copy(tin_hbm.at[pl.ds(base, _RB)], bins.at[0], sin)

    for i in range(n_it):
        b = i & 1
        nb = 1 - b
        if i + 1 < n_it:
            ins[nb] = pltpu.async_copy(
                tin_hbm.at[pl.ds(base + (i + 1) * _RB, _RB)], bins.at[nb], sin)
        ins[b].wait()
        if i >= 2:
            outs[b].wait()
        interleave(bins.at[b], bouts.at[b])
        outs[b] = pltpu.async_copy(
            bouts.at[b], tout_hbm.at[pl.ds((base + i * _RB) * 4, _RB * 4)], sout)
    outs[(n_it - 1) & 1].wait()
    outs[n_it & 1].wait()


@functools.lru_cache(maxsize=None)
def _build_repack():
    return pl.kernel(
        _repack_body,
        out_type=jax.ShapeDtypeStruct((4 * _M,), jnp.float32),
        mesh=plsc.VectorSubcoreMesh(core_axis_name="c", subcore_axis_name="s",
                                    num_cores=_NC, num_subcores=_NS),
        scratch_types=[
            pltpu.VMEM((2, _RB), jnp.float32),
            pltpu.VMEM((2, 4 * _RB), jnp.float32),
            pltpu.SemaphoreType.DMA,
            pltpu.SemaphoreType.DMA,
        ],
        compiler_params=pltpu.CompilerParams(use_tc_tiling_on_sc=False,
                                             needs_layout_passes=False),
    )


def _sc_body(dirs_hbm, table_hbm, out_hbm, dbuf, idxb, wb, gb, feat,
             sem0, sem1):
    wid = lax.axis_index("s") * _NC + lax.axis_index("c")
    iota = lax.iota(jnp.int32, _LANES)
    zero_i = jnp.zeros((_LANES,), jnp.int32)
    one_i = jnp.ones((_LANES,), jnp.int32)
    sems = [sem0, sem1]

    def chunk_body(ch, carry):
        base = wid * _PW + ch * _C
        pltpu.sync_copy(dirs_hbm.at[pl.ds(base * 3, 3 * _C)], dbuf)

        def grpd(g, _):
            s = pl.ds(g * _LANES, _LANES)
            pidx3 = (iota + g * _LANES) * 3
            feat[32, s] = plsc.load_gather(dbuf, [pidx3])
            feat[33, s] = plsc.load_gather(dbuf, [pidx3 + 1])
            feat[34, s] = plsc.load_gather(dbuf, [pidx3 + 2])
            return _

        lax.fori_loop(0, _G, grpd, 0)

        def make_grp1(lvl):
            res = float(_RES[lvl])
            lvl_off = lvl * _T
            b = lvl & 1

            def grp1(g, _, res=res, lvl_off=lvl_off, b=b):
                s = pl.ds(g * _LANES, _LANES)
                pidx3 = (iota + g * _LANES) * 3
                px = (plsc.load_gather(dbuf, [pidx3]) * 0.49 + 0.49) * res
                py = (plsc.load_gather(dbuf, [pidx3 + 1]) * 0.49 + 0.49) * res
                pz = (plsc.load_gather(dbuf, [pidx3 + 2]) * 0.49 + 0.49) * res
                xi = px.astype(jnp.int32)
                yi = py.astype(jnp.int32)
                zi = pz.astype(jnp.int32)
                wx = px - xi.astype(jnp.float32)
                wy = py - yi.astype(jnp.float32)
                wz = pz - zi.astype(jnp.float32)
                wx = wx * wx * (3.0 - 2.0 * wx)
                wy = wy * wy * (3.0 - 2.0 * wy)
                wz = wz * wz * (3.0 - 2.0 * wz)
                x0 = xi.astype(jnp.uint32)
                y0 = yi.astype(jnp.uint32)
                z0 = zi.astype(jnp.uint32)
                hx = [x0, x0 + jnp.uint32(1)]
                hy0 = y0 * jnp.uint32(_P2)
                hz0 = z0 * jnp.uint32(_P3)
                hy = [hy0, hy0 + jnp.uint32(_P2)]
                hz = [hz0, hz0 + jnp.uint32(_P3)]
                wx_ = [1.0 - wx, wx]
                syz = [(1.0 - wy) * (1.0 - wz), wy * (1.0 - wz),
                       (1.0 - wy) * wz, wy * wz]
                row = idxb.at[b]
                for c in range(8):
                    bx, by, bz = c & 1, (c >> 1) & 1, (c >> 2) & 1
                    h = hx[bx] ^ hy[by] ^ hz[bz]
                    idx = (h & jnp.uint32(_TMASK)).astype(jnp.int32) + lvl_off
                    row[pl.ds(g * 8 * _LANES + c * _LANES, _LANES)] = idx
                    wb[b, c, s] = wx_[bx] * syz[by + 2 * bz]
                return _

            return grp1

        def fire(lvl):
            b = lvl & 1
            return [pltpu.async_copy(table_hbm.at[idxb.at[b]],
                                     gb.at[b], sems[b])]

        def make_grp2(lvl):
            b = lvl & 1

            def grp2(g, _, lvl=lvl, b=b):
                s = pl.ds(g * _LANES, _LANES)
                rows = gb.at[b]
                f0 = jnp.zeros((_LANES,), jnp.float32)
                f1 = jnp.zeros((_LANES,), jnp.float32)
                for c in range(8):
                    ridx = iota + g * 8 * _LANES + c * _LANES
                    g0 = plsc.load_gather(rows, [ridx, zero_i])
                    g1 = plsc.load_gather(rows, [ridx, one_i])
                    wc = wb[b, c, s]
                    f0 = f0 + wc * g0
                    f1 = f1 + wc * g1
                feat[2 * lvl, s] = f0
                feat[2 * lvl + 1, s] = f1
                return _

            return grp2

        def run(fn):
            plsc.parallel_loop(0, _G, 1, unroll=2)(
                lambda g: fn(g, 0) and None)

        run(make_grp1(0))
        cps = fire(0)
        for lvl in range(1, _N_LEVELS):
            run(make_grp1(lvl))
            cps_new = fire(lvl)
            for cp in cps:
                cp.wait()
            run(make_grp2(lvl - 1))
            cps = cps_new
        for cp in cps:
            cp.wait()
        run(make_grp2(_N_LEVELS - 1))

        pltpu.sync_copy(feat, out_hbm.at[:, pl.ds(base, _C)])
        return carry

    lax.fori_loop(0, _CHUNKS, chunk_body, 0)


@functools.lru_cache(maxsize=None)
def _build_sc_features():
    return pl.kernel(
        _sc_body,
        out_type=jax.ShapeDtypeStruct((_F, _N), jnp.float32),
        mesh=plsc.VectorSubcoreMesh(core_axis_name="c", subcore_axis_name="s",
                                    num_cores=_NC, num_subcores=_NS),
        scratch_types=[
            pltpu.VMEM((3 * _C,), jnp.float32),
            pltpu.VMEM((2, _G * 8 * _LANES), jnp.int32),
            pltpu.VMEM((2, 8, _C), jnp.float32),
            pltpu.VMEM((2, _G * 8 * _LANES, 8), jnp.float32),
            pltpu.VMEM((_F, _C), jnp.float32),
            pltpu.SemaphoreType.DMA,
            pltpu.SemaphoreType.DMA,
        ],
        compiler_params=pltpu.CompilerParams(use_tc_tiling_on_sc=False,
                                             needs_layout_passes=False),
    )


_B = 2048


def _mlp_body(feat_ref, w0_ref, b0_ref, w1_ref, b1_ref,
              w2_ref, b2_ref, out_ref):
    fb = feat_ref[...]
    dn = (((0,), (0,)), ((), ()))
    h = lax.dot_general(fb, w0_ref[...], dn, preferred_element_type=jnp.float32)
    h = jnp.maximum(h + b0_ref[...], 0.0)
    h = jnp.maximum(
        jnp.dot(h, w1_ref[...], preferred_element_type=jnp.float32) + b1_ref[...],
        0.0)
    o = jnp.sum(h * w2_ref[...], axis=1) + b2_ref[0, 0] + 1.0
    dist = jnp.maximum(o, 0.0) + jnp.log(1.0 + jnp.exp(-jnp.abs(o)))
    out_ref[...] = dist


_mlp = pl.pallas_call(
    _mlp_body,
    grid=(_N // _B,),
    in_specs=[
        pl.BlockSpec((_F, _B), lambda i: (0, i)),
        pl.BlockSpec((_F, 64), lambda i: (0, 0)),
        pl.BlockSpec((1, 64), lambda i: (0, 0)),
        pl.BlockSpec((64, 64), lambda i: (0, 0)),
        pl.BlockSpec((1, 64), lambda i: (0, 0)),
        pl.BlockSpec((1, 64), lambda i: (0, 0)),
        pl.BlockSpec(memory_space=pltpu.SMEM),
    ],
    out_specs=pl.BlockSpec((_B,), lambda i: (i,)),
    out_shape=jax.ShapeDtypeStruct((_N,), jnp.float32),
)


def kernel(directions, table, W0, b0, W1, b1, W2, b2):
    table_fm = table.reshape(_N_LEVELS, _T // 128, 128, 2)
    table_fm = table_fm.transpose(0, 1, 3, 2).reshape(_M)
    table_ew = _build_repack()(table_fm).reshape(_N_LEVELS * _T, 8)
    feat_t = _build_sc_features()(directions.reshape(3 * _N), table_ew)
    w0r = jnp.concatenate([W0[3:], W0[:3], jnp.zeros((1, 64), jnp.float32)])
    return _mlp(feat_t, w0r, b0.reshape(1, 64), W1,
                b1.reshape(1, 64), W2.reshape(1, 64), b2.reshape(1, 1))


# speedup vs baseline: 1.0772x; 1.0772x over previous
import functools

import numpy as np
import jax
import jax.numpy as jnp
from jax import lax
from jax.experimental import pallas as pl
from jax.experimental.pallas import tpu as pltpu
from jax.experimental.pallas import tpu_sc as plsc

_N_LEVELS = 16
_LOG2_T = 19
_T = 1 << _LOG2_T
_TMASK = _T - 1
_BASE_RES = 16
_FINE_RES = 2048
_N = 262144
_SCALE = float(np.exp(np.log(_FINE_RES / _BASE_RES) / (_N_LEVELS - 1)))
_RES = [int(np.floor(_BASE_RES * _SCALE ** l)) for l in range(_N_LEVELS)]
_P2 = 2654435761
_P3 = 805459861

_NC, _NS, _LANES = 2, 16, 16
_NW = _NC * _NS
_C = 256
_G = _C // _LANES
_PW = _N // _NW
_CHUNKS = _PW // _C

_F = 2 * _N_LEVELS + 4
_M = _N_LEVELS * _T * 2
_PT = _M // _NW
_RB = 8192


def _repack_body(tin_hbm, tout_hbm, bins, bouts, sin, sout):
    wid = lax.axis_index("s") * _NC + lax.axis_index("c")
    base = wid * _PT
    iota8 = lax.iota(jnp.int32, _LANES) * 8
    n_it = _PT // _RB

    def interleave(d, bout):
        def tp(k, c2):
            for q in range(8):
                f0 = d[pl.ds(k * 256 + q * 16, 16)]
                f1 = d[pl.ds(k * 256 + 128 + q * 16, 16)]
                plsc.store_scatter(bout, [(k * 128 + q * 16) * 8 + iota8], f0)
                plsc.store_scatter(bout, [(k * 128 + q * 16) * 8 + 1 + iota8], f1)
            return c2

        lax.fori_loop(0, _RB // 256, tp, 0)

    ins = [None, None]
    outs = [None, None]
    ins[0] = pltpu.async_copy(tin_hbm.at[pl.ds(base, _RB)], bins.at[0], sin)

    for i in range(n_it):
        b = i & 1
        nb = 1 - b
        if i + 1 < n_it:
            ins[nb] = pltpu.async_copy(
                tin_hbm.at[pl.ds(base + (i + 1) * _RB, _RB)], bins.at[nb], sin)
        ins[b].wait()
        if i >= 2:
            outs[b].wait()
        interleave(bins.at[b], bouts.at[b])
        outs[b] = pltpu.async_copy(
            bouts.at[b], tout_hbm.at[pl.ds((base + i * _RB) * 4, _RB * 4)], sout)
    outs[(n_it - 1) & 1].wait()
    outs[n_it & 1].wait()


@functools.lru_cache(maxsize=None)
def _build_repack():
    return pl.kernel(
        _repack_body,
        out_type=jax.ShapeDtypeStruct((4 * _M,), jnp.float32),
        mesh=plsc.VectorSubcoreMesh(core_axis_name="c", subcore_axis_name="s",
                                    num_cores=_NC, num_subcores=_NS),
        scratch_types=[
            pltpu.VMEM((2, _RB), jnp.float32),
            pltpu.VMEM((2, 4 * _RB), jnp.float32),
            pltpu.SemaphoreType.DMA,
            pltpu.SemaphoreType.DMA,
        ],
        compiler_params=pltpu.CompilerParams(use_tc_tiling_on_sc=False,
                                             needs_layout_passes=False),
    )


def _sc_body(dirs_hbm, table_hbm, out_hbm, dbuf, idxb, wb, gb, feat,
             sem0, sem1):
    wid = lax.axis_index("s") * _NC + lax.axis_index("c")
    iota = lax.iota(jnp.int32, _LANES)
    zero_i = jnp.zeros((_LANES,), jnp.int32)
    one_i = jnp.ones((_LANES,), jnp.int32)
    sems = [sem0, sem1]

    def chunk_body(ch, carry):
        base = wid * _PW + ch * _C
        pltpu.sync_copy(dirs_hbm.at[pl.ds(base * 3, 3 * _C)], dbuf)

        def grpd(g, _):
            s = pl.ds(g * _LANES, _LANES)
            pidx3 = (iota + g * _LANES) * 3
            feat[32, s] = plsc.load_gather(dbuf, [pidx3])
            feat[33, s] = plsc.load_gather(dbuf, [pidx3 + 1])
            feat[34, s] = plsc.load_gather(dbuf, [pidx3 + 2])
            return _

        lax.fori_loop(0, _G, grpd, 0)

        def make_grp1(lvl):
            res = float(_RES[lvl])
            lvl_off = lvl * _T
            b = lvl & 1

            def grp1(g, _, res=res, lvl_off=lvl_off, b=b):
                s = pl.ds(g * _LANES, _LANES)
                pidx3 = (iota + g * _LANES) * 3
                px = (plsc.load_gather(dbuf, [pidx3]) * 0.49 + 0.49) * res
                py = (plsc.load_gather(dbuf, [pidx3 + 1]) * 0.49 + 0.49) * res
                pz = (plsc.load_gather(dbuf, [pidx3 + 2]) * 0.49 + 0.49) * res
                xi = px.astype(jnp.int32)
                yi = py.astype(jnp.int32)
                zi = pz.astype(jnp.int32)
                wx = px - xi.astype(jnp.float32)
                wy = py - yi.astype(jnp.float32)
                wz = pz - zi.astype(jnp.float32)
                wx = wx * wx * (3.0 - 2.0 * wx)
                wy = wy * wy * (3.0 - 2.0 * wy)
                wz = wz * wz * (3.0 - 2.0 * wz)
                x0 = xi.astype(jnp.uint32)
                y0 = yi.astype(jnp.uint32)
                z0 = zi.astype(jnp.uint32)
                hx = [x0, x0 + jnp.uint32(1)]
                hy0 = y0 * jnp.uint32(_P2)
                hz0 = z0 * jnp.uint32(_P3)
                hy = [hy0, hy0 + jnp.uint32(_P2)]
                hz = [hz0, hz0 + jnp.uint32(_P3)]
                wx_ = [1.0 - wx, wx]
                syz = [(1.0 - wy) * (1.0 - wz), wy * (1.0 - wz),
                       (1.0 - wy) * wz, wy * wz]
                row = idxb.at[b]
                for c in range(8):
                    bx, by, bz = c & 1, (c >> 1) & 1, (c >> 2) & 1
                    h = hx[bx] ^ hy[by] ^ hz[bz]
                    idx = (h & jnp.uint32(_TMASK)).astype(jnp.int32) + lvl_off
                    row[pl.ds(g * 8 * _LANES + c * _LANES, _LANES)] = idx
                    wb[b, c, s] = wx_[bx] * syz[by + 2 * bz]
                return _

            return grp1

        def fire(lvl):
            b = lvl & 1
            return [pltpu.async_copy(table_hbm.at[idxb.at[b]],
                                     gb.at[b], sems[b])]

        def make_grp2(lvl):
            b = lvl & 1

            def grp2(g, _, lvl=lvl, b=b):
                s = pl.ds(g * _LANES, _LANES)
                rows = gb.at[b]
                f0 = jnp.zeros((_LANES,), jnp.float32)
                f1 = jnp.zeros((_LANES,), jnp.float32)
                for c in range(8):
                    ridx = iota + g * 8 * _LANES + c * _LANES
                    g0 = plsc.load_gather(rows, [ridx, zero_i])
                    g1 = plsc.load_gather(rows, [ridx, one_i])
                    wc = wb[b, c, s]
                    f0 = f0 + wc * g0
                    f1 = f1 + wc * g1
                feat[2 * lvl, s] = f0
                feat[2 * lvl + 1, s] = f1
                return _

            return grp2

        def run(fn):
            plsc.parallel_loop(0, _G, 1, unroll=2)(
                lambda g: fn(g, 0) and None)

        run(make_grp1(0))
        cps = fire(0)
        for lvl in range(1, _N_LEVELS):
            run(make_grp1(lvl))
            cps_new = fire(lvl)
            for cp in cps:
                cp.wait()
            run(make_grp2(lvl - 1))
            cps = cps_new
        for cp in cps:
            cp.wait()
        run(make_grp2(_N_LEVELS - 1))

        pltpu.sync_copy(feat, out_hbm.at[:, pl.ds(base, _C)])
        return carry

    lax.fori_loop(0, _CHUNKS, chunk_body, 0)


@functools.lru_cache(maxsize=None)
def _build_sc_features():
    return pl.kernel(
        _sc_body,
        out_type=jax.ShapeDtypeStruct((_F, _N), jnp.float32),
        mesh=plsc.VectorSubcoreMesh(core_axis_name="c", subcore_axis_name="s",
                                    num_cores=_NC, num_subcores=_NS),
        scratch_types=[
            pltpu.VMEM((3 * _C,), jnp.float32),
            pltpu.VMEM((2, _G * 8 * _LANES), jnp.int32),
            pltpu.VMEM((2, 8, _C), jnp.float32),
            pltpu.VMEM((2, _G * 8 * _LANES, 8), jnp.float32),
            pltpu.VMEM((_F, _C), jnp.float32),
            pltpu.SemaphoreType.DMA,
            pltpu.SemaphoreType.DMA,
        ],
        compiler_params=pltpu.CompilerParams(use_tc_tiling_on_sc=False,
                                             needs_layout_passes=False),
    )


_B = 2048


def _mlp_body(feat_ref, w0_ref, b0_ref, w1_ref, b1_ref,
              w2_ref, b2_ref, out_ref):
    fb = feat_ref[...]
    dn = (((0,), (0,)), ((), ()))
    h = lax.dot_general(w0_ref[...], fb, dn, preferred_element_type=jnp.float32)
    h = jnp.maximum(h + b0_ref[...], 0.0)
    h = lax.dot_general(w1_ref[...], h, dn, preferred_element_type=jnp.float32)
    h = jnp.maximum(h + b1_ref[...], 0.0)
    o = lax.dot_general(w2_ref[...], h, dn,
                        preferred_element_type=jnp.float32)
    o = o + (b2_ref[0, 0] + 1.0)
    dist = jnp.maximum(o, 0.0) + jnp.log(1.0 + jnp.exp(-jnp.abs(o)))
    out_ref[...] = dist


_mlp = pl.pallas_call(
    _mlp_body,
    grid=(_N // _B,),
    in_specs=[
        pl.BlockSpec((_F, _B), lambda i: (0, i)),
        pl.BlockSpec((_F, 64), lambda i: (0, 0)),
        pl.BlockSpec((64, 1), lambda i: (0, 0)),
        pl.BlockSpec((64, 64), lambda i: (0, 0)),
        pl.BlockSpec((64, 1), lambda i: (0, 0)),
        pl.BlockSpec((64, 1), lambda i: (0, 0)),
        pl.BlockSpec(memory_space=pltpu.SMEM),
    ],
    out_specs=pl.BlockSpec((1, _B), lambda i: (0, i)),
    out_shape=jax.ShapeDtypeStruct((1, _N), jnp.float32),
)


def kernel(directions, table, W0, b0, W1, b1, W2, b2):
    table_fm = table.reshape(_N_LEVELS, _T // 128, 128, 2)
    table_fm = table_fm.transpose(0, 1, 3, 2).reshape(_M)
    table_ew = _build_repack()(table_fm).reshape(_N_LEVELS * _T, 8)
    feat_t = _build_sc_features()(directions.reshape(3 * _N), table_ew)
    w0r = jnp.concatenate([W0[3:], W0[:3], jnp.zeros((1, 64), jnp.float32)])
    return _mlp(feat_t, w0r, b0.reshape(64, 1), W1,
                b1.reshape(64, 1), W2, b2.reshape(1, 1)).reshape(_N)
